# baseline (speedup 1.0000x reference)
import jax
import jax.numpy as jnp
from jax.experimental import pallas as pl
from jax.experimental.pallas import tpu as pltpu

_TAU_INV = 2.0
_KC = 7
_NC = 4
_NBUF = 6


def _body(l_ref, u_hbm, o_ref, a_ref, buf, sems):
    g = pl.program_id(0)
    n = pl.num_programs(0)
    kc = jax.lax.rem(g, _NC)
    slot = jax.lax.rem(g, _NBUF)

    @pl.when(g == 0)
    def _prologue():
        for j in range(_NBUF):
            pltpu.make_async_copy(u_hbm.at[j], buf.at[j], sems.at[j]).start()

    pltpu.make_async_copy(u_hbm.at[g], buf.at[slot], sems.at[slot]).wait()

    @pl.when(kc == 0)
    def _row_setup():
        a_ref[...] = jnp.exp(l_ref[0] * _TAU_INV)

    u = buf[slot]
    t = jnp.log(u)
    aw = a_ref[...] / (t * t)
    s = jnp.sum(aw, axis=1, keepdims=True)
    m = jnp.max(aw * (1.0 / s), axis=0, keepdims=True)

    @pl.when(kc == 0)
    def _init_out():
        o_ref[0] = m

    @pl.when(kc != 0)
    def _acc_out():
        o_ref[0] = jnp.maximum(o_ref[0], m)

    g2 = g + _NBUF

    @pl.when(g2 < n)
    def _refill():
        slot2 = jax.lax.rem(g2, _NBUF)
        pltpu.make_async_copy(
            u_hbm.at[g2], buf.at[slot2], sems.at[slot2]).start()


def kernel(logits, uniform):
    B, K, D = uniform.shape
    nchunks = B * _NC
    out = pl.pallas_call(
        _body,
        grid=(nchunks,),
        in_specs=[
            pl.BlockSpec((1, 1, D), lambda g: (g // _NC, 0, 0)),
            pl.BlockSpec(memory_space=pltpu.HBM),
        ],
        out_specs=pl.BlockSpec((1, 1, D), lambda g: (g // _NC, 0, 0)),
        out_shape=jax.ShapeDtypeStruct((B, 1, D), jnp.float32),
        scratch_shapes=[
            pltpu.VMEM((1, D), jnp.float32),
            pltpu.VMEM((_NBUF, _KC, D), jnp.float32),
            pltpu.SemaphoreType.DMA((_NBUF,)),
        ],
        compiler_params=pltpu.CompilerParams(
            dimension_semantics=("arbitrary",)),
    )(logits.reshape(B, 1, D), uniform.reshape(nchunks, _KC, D))
    return out.reshape(B, D)


# speedup vs baseline: 1.7675x; 1.7675x over previous
import jax
import jax.numpy as jnp
from jax.experimental import pallas as pl
from jax.experimental.pallas import tpu as pltpu

_TAU_INV = 2.0
_NBUF = 4
_NSPLIT = 4
_NCHUNK = 4


def _start_row(u_hbm, buf, sems, row, slot, D):
    Ds = D // _NSPLIT
    for j in range(_NSPLIT):
        pltpu.make_async_copy(
            u_hbm.at[row, :, pl.ds(j * Ds, Ds)],
            buf.at[slot, :, pl.ds(j * Ds, Ds)],
            sems.at[slot, j],
        ).start()


def _wait_row(u_hbm, buf, sems, row, slot, D):
    Ds = D // _NSPLIT
    for j in range(_NSPLIT):
        pltpu.make_async_copy(
            u_hbm.at[row, :, pl.ds(j * Ds, Ds)],
            buf.at[slot, :, pl.ds(j * Ds, Ds)],
            sems.at[slot, j],
        ).wait()


def _body(l_ref, u_hbm, o_ref, buf, sems):
    b = pl.program_id(0)
    n = pl.num_programs(0)
    D = u_hbm.shape[2]
    slot = jax.lax.rem(b, _NBUF)

    @pl.when(b == 0)
    def _prologue():
        for j in range(_NBUF):
            _start_row(u_hbm, buf, sems, j, j, D)

    _wait_row(u_hbm, buf, sems, b, slot, D)

    Dc = D // _NCHUNK
    ars = []
    s = None
    for i in range(_NCHUNK):
        a = jnp.exp(l_ref[0, :, i * Dc:(i + 1) * Dc] * _TAU_INV)
        t = jnp.log(buf[slot, :, i * Dc:(i + 1) * Dc])
        ar = a / (t * t)
        ars.append(ar)
        p = jnp.sum(ar, axis=1, keepdims=True)
        s = p if s is None else s + p
    r = 1.0 / s
    for i, ar in enumerate(ars):
        o_ref[0, :, i * Dc:(i + 1) * Dc] = jnp.max(
            ar * r, axis=0, keepdims=True)

    b2 = b + _NBUF

    @pl.when(b2 < n)
    def _refill():
        _start_row(u_hbm, buf, sems, b2, jax.lax.rem(b2, _NBUF), D)


def kernel(logits, uniform):
    B, K, D = uniform.shape
    out = pl.pallas_call(
        _body,
        grid=(B,),
        in_specs=[
            pl.BlockSpec((1, 1, D), lambda b: (b, 0, 0)),
            pl.BlockSpec(memory_space=pltpu.HBM),
        ],
        out_specs=pl.BlockSpec((1, 1, D), lambda b: (b, 0, 0)),
        out_shape=jax.ShapeDtypeStruct((B, 1, D), jnp.float32),
        scratch_shapes=[
            pltpu.VMEM((_NBUF, K, D), jnp.float32),
            pltpu.SemaphoreType.DMA((_NBUF, _NSPLIT)),
        ],
        compiler_params=pltpu.CompilerParams(
            dimension_semantics=("arbitrary",)),
    )(logits.reshape(B, 1, D), uniform)
    return out.reshape(B, D)
